# initial kernel scaffold (unmeasured)
import jax
import jax.numpy as jnp
from jax import lax
from jax.experimental import pallas as pl
from jax.experimental.pallas import tpu as pltpu

N_DEV = 4
M_PER = 2048
K = 8192
N_PER = 1024
HALF_K = K // 2
BM = 512
N_TILES = M_PER // BM


def kernel(x, w_mat):
    x = x.astype(jnp.bfloat16)
    w = w_mat.astype(jnp.bfloat16)

    def body(x_ref, w_ref, y_ref, xg_ref, xt, yt,
             load_sems, store_sems, send_sems, recv_sems):
        me = lax.axis_index("i")
        left = lax.rem(me + (N_DEV - 1), N_DEV)
        right = lax.rem(me + 1, N_DEV)
        diag = lax.rem(me + 2, N_DEV)

        barrier = pltpu.get_barrier_semaphore()
        pl.semaphore_signal(barrier, inc=1, device_id=(left,),
                            device_id_type=pl.DeviceIdType.MESH)
        pl.semaphore_signal(barrier, inc=1, device_id=(right,),
                            device_id_type=pl.DeviceIdType.MESH)
        pl.semaphore_wait(barrier, 2)

        def rdma(src, dst, i, dev):
            return pltpu.make_async_remote_copy(
                src_ref=src, dst_ref=dst,
                send_sem=send_sems.at[i], recv_sem=recv_sems.at[i],
                device_id=(dev,), device_id_type=pl.DeviceIdType.MESH,
            )

        slot0 = xg_ref.at[pl.ds(0, M_PER), :]
        slot1 = xg_ref.at[pl.ds(M_PER, M_PER), :]
        rdma_a = rdma(x_ref, slot0, 0, right)
        rdma_b = rdma(x_ref, slot1, 1, left)
        rdma_c = rdma(xg_ref.at[pl.ds(0, M_PER), pl.ds(0, HALF_K)],
                      xg_ref.at[pl.ds(2 * M_PER, M_PER), pl.ds(0, HALF_K)],
                      2, right)
        rdma_d = rdma(xg_ref.at[pl.ds(M_PER, M_PER), pl.ds(HALF_K, HALF_K)],
                      xg_ref.at[pl.ds(2 * M_PER, M_PER), pl.ds(HALF_K, HALF_K)],
                      3, left)

        def gemm_chunk(src_ref, src_row0, out_row0):
            def load(i):
                return pltpu.make_async_copy(
                    src_ref.at[pl.ds(src_row0 + i * BM, BM), :],
                    xt.at[i % 2], load_sems.at[i % 2])

            def store(i):
                return pltpu.make_async_copy(
                    yt.at[i % 2],
                    y_ref.at[pl.ds(out_row0 + i * BM, BM), :],
                    store_sems.at[i % 2])

            load(0).start()
            for i in range(N_TILES):
                if i + 1 < N_TILES:
                    load(i + 1).start()
                load(i).wait()
                acc = jnp.dot(xt[i % 2], w_ref[...],
                              preferred_element_type=jnp.float32)
                if i >= 2:
                    store(i - 2).wait()
                yt[i % 2] = jnp.maximum(acc, 0.0)
                store(i).start()
            store(N_TILES - 2).wait()
            store(N_TILES - 1).wait()

        rdma_a.start()
        rdma_b.start()
        gemm_chunk(x_ref, 0, me * M_PER)

        rdma_a.wait_recv()
        rdma_c.start()
        gemm_chunk(xg_ref, 0, left * M_PER)

        rdma_b.wait_recv()
        rdma_d.start()
        gemm_chunk(xg_ref, M_PER, right * M_PER)

        rdma_c.wait_recv()
        rdma_d.wait_recv()
        gemm_chunk(xg_ref, 2 * M_PER, diag * M_PER)

        rdma_a.wait_send()
        rdma_b.wait_send()
        rdma_c.wait_send()
        rdma_d.wait_send()

    y, _ = pl.pallas_call(
        body,
        out_shape=[
            jax.ShapeDtypeStruct((N_DEV * M_PER, N_PER), jnp.float32),
            jax.ShapeDtypeStruct((3 * M_PER, K), jnp.bfloat16),
        ],
        in_specs=[
            pl.BlockSpec(memory_space=pltpu.ANY),
            pl.BlockSpec(memory_space=pltpu.VMEM),
        ],
        out_specs=[
            pl.BlockSpec(memory_space=pltpu.ANY),
            pl.BlockSpec(memory_space=pltpu.ANY),
        ],
        scratch_shapes=[
            pltpu.VMEM((2, BM, K), jnp.bfloat16),
            pltpu.VMEM((2, BM, N_PER), jnp.float32),
            pltpu.SemaphoreType.DMA((2,)),
            pltpu.SemaphoreType.DMA((2,)),
            pltpu.SemaphoreType.DMA((4,)),
            pltpu.SemaphoreType.DMA((4,)),
        ],
        compiler_params=pltpu.CompilerParams(collective_id=0),
    )(x, w)
    return y


# baseline (device time: 701589 ns/iter reference)
import jax
import jax.numpy as jnp
from jax import lax
from jax.experimental import pallas as pl
from jax.experimental.pallas import tpu as pltpu

N_DEV = 4
M_PER = 2048
K = 8192
N_PER = 1024
HALF_M = M_PER // 2
BM = 512
N_TILES = M_PER // BM


def kernel(x, w_mat):
    x = x.astype(jnp.bfloat16)
    w = w_mat.astype(jnp.bfloat16)

    def body(x_ref, w_ref, y_ref, xg_ref, xt, yt,
             load_sems, store_sems, send_sems, recv_sems):
        me = lax.axis_index("i")
        left = lax.rem(me + (N_DEV - 1), N_DEV)
        right = lax.rem(me + 1, N_DEV)
        diag = lax.rem(me + 2, N_DEV)

        barrier = pltpu.get_barrier_semaphore()
        pl.semaphore_signal(barrier, inc=1, device_id=(left,),
                            device_id_type=pl.DeviceIdType.MESH)
        pl.semaphore_signal(barrier, inc=1, device_id=(right,),
                            device_id_type=pl.DeviceIdType.MESH)
        pl.semaphore_wait(barrier, 2)

        def rdma(src, dst, i, dev):
            return pltpu.make_async_remote_copy(
                src_ref=src, dst_ref=dst,
                send_sem=send_sems.at[i], recv_sem=recv_sems.at[i],
                device_id=(dev,), device_id_type=pl.DeviceIdType.MESH,
            )

        slot0 = xg_ref.at[pl.ds(0, M_PER), :]
        slot1 = xg_ref.at[pl.ds(M_PER, M_PER), :]
        rdma_a = rdma(x_ref, slot0, 0, right)
        rdma_b = rdma(x_ref, slot1, 1, left)
        rdma_c = rdma(xg_ref.at[pl.ds(0, HALF_M), :],
                      xg_ref.at[pl.ds(2 * M_PER, HALF_M), :],
                      2, right)
        rdma_d = rdma(xg_ref.at[pl.ds(M_PER + HALF_M, HALF_M), :],
                      xg_ref.at[pl.ds(2 * M_PER + HALF_M, HALF_M), :],
                      3, left)

        def gemm_chunk(src_ref, src_row0, out_row0):
            def load(i, slot):
                return pltpu.make_async_copy(
                    src_ref.at[pl.ds(src_row0 + i * BM, BM), :],
                    xt.at[slot], load_sems.at[slot])

            def store(i, slot):
                return pltpu.make_async_copy(
                    yt.at[slot],
                    y_ref.at[pl.ds(out_row0 + i * BM, BM), :],
                    store_sems.at[slot])

            load(0, 0).start()

            def tile_body(i, carry):
                slot = lax.rem(i, 2)

                @pl.when(i + 1 < N_TILES)
                def _():
                    load(i + 1, lax.rem(i + 1, 2)).start()

                load(i, slot).wait()
                acc = jnp.dot(xt[slot], w_ref[...],
                              preferred_element_type=jnp.float32)

                @pl.when(i >= 2)
                def _():
                    store(i - 2, slot).wait()

                yt[slot] = jnp.maximum(acc, 0.0)
                store(i, slot).start()
                return carry

            lax.fori_loop(0, N_TILES, tile_body, 0)
            store(N_TILES - 2, lax.rem(N_TILES - 2, 2)).wait()
            store(N_TILES - 1, lax.rem(N_TILES - 1, 2)).wait()

        rdma_a.start()
        rdma_b.start()
        gemm_chunk(x_ref, 0, me * M_PER)

        rdma_a.wait_recv()
        rdma_c.start()
        gemm_chunk(xg_ref, 0, left * M_PER)

        rdma_b.wait_recv()
        rdma_d.start()
        gemm_chunk(xg_ref, M_PER, right * M_PER)

        rdma_c.wait_recv()
        rdma_d.wait_recv()
        gemm_chunk(xg_ref, 2 * M_PER, diag * M_PER)

        rdma_a.wait_send()
        rdma_b.wait_send()
        rdma_c.wait_send()
        rdma_d.wait_send()

    y, _ = pl.pallas_call(
        body,
        out_shape=[
            jax.ShapeDtypeStruct((N_DEV * M_PER, N_PER), jnp.float32),
            jax.ShapeDtypeStruct((3 * M_PER, K), jnp.bfloat16),
        ],
        in_specs=[
            pl.BlockSpec(memory_space=pl.ANY),
            pl.BlockSpec(memory_space=pltpu.VMEM),
        ],
        out_specs=[
            pl.BlockSpec(memory_space=pl.ANY),
            pl.BlockSpec(memory_space=pl.ANY),
        ],
        scratch_shapes=[
            pltpu.VMEM((2, BM, K), jnp.bfloat16),
            pltpu.VMEM((2, BM, N_PER), jnp.float32),
            pltpu.SemaphoreType.DMA((2,)),
            pltpu.SemaphoreType.DMA((2,)),
            pltpu.SemaphoreType.DMA((4,)),
            pltpu.SemaphoreType.DMA((4,)),
        ],
        compiler_params=pltpu.CompilerParams(collective_id=0),
    )(x, w)
    return y


# device time: 672288 ns/iter; 1.0436x vs baseline; 1.0436x over previous
import jax
import jax.numpy as jnp
from jax import lax
from jax.experimental import pallas as pl
from jax.experimental.pallas import tpu as pltpu

N_DEV = 4
M_PER = 2048
K = 8192
N_PER = 1024
HALF_M = M_PER // 2
BM = 256
SUB = 512
N_TILES = M_PER // BM


def kernel(x, w_mat):
    x = x.astype(jnp.bfloat16)
    w = w_mat.astype(jnp.bfloat16)

    def body(x_ref, w_ref, y_ref, xg_ref, xt, yt,
             load_sems, store_sems, send_sems, recv_sems):
        me = lax.axis_index("i")
        left = lax.rem(me + (N_DEV - 1), N_DEV)
        right = lax.rem(me + 1, N_DEV)
        diag = lax.rem(me + 2, N_DEV)

        barrier = pltpu.get_barrier_semaphore()
        pl.semaphore_signal(barrier, inc=1, device_id=(left,),
                            device_id_type=pl.DeviceIdType.MESH)
        pl.semaphore_signal(barrier, inc=1, device_id=(right,),
                            device_id_type=pl.DeviceIdType.MESH)
        pl.semaphore_wait(barrier, 2)

        def rdma(src, dst, i, dev):
            return pltpu.make_async_remote_copy(
                src_ref=src, dst_ref=dst,
                send_sem=send_sems.at[i], recv_sem=recv_sems.at[i],
                device_id=(dev,), device_id_type=pl.DeviceIdType.MESH,
            )

        slot0 = xg_ref.at[pl.ds(0, M_PER), :]
        slot1 = xg_ref.at[pl.ds(M_PER, M_PER), :]
        rdma_a = rdma(x_ref, slot0, 0, right)
        rdma_b = rdma(x_ref, slot1, 1, left)
        rdma_c = [
            rdma(xg_ref.at[pl.ds(j * SUB, SUB), :],
                 xg_ref.at[pl.ds(2 * M_PER + j * SUB, SUB), :],
                 2 + j, right)
            for j in range(HALF_M // SUB)
        ]
        rdma_d = [
            rdma(xg_ref.at[pl.ds(M_PER + HALF_M + j * SUB, SUB), :],
                 xg_ref.at[pl.ds(2 * M_PER + HALF_M + j * SUB, SUB), :],
                 2 + HALF_M // SUB + j, left)
            for j in range(HALF_M // SUB)
        ]

        def gemm_chunk(src_ref, src_row0, out_row0, n_tiles=N_TILES):
            def load(i, slot):
                return pltpu.make_async_copy(
                    src_ref.at[pl.ds(src_row0 + i * BM, BM), :],
                    xt.at[slot], load_sems.at[slot])

            def store(i, slot):
                return pltpu.make_async_copy(
                    yt.at[slot],
                    y_ref.at[pl.ds(out_row0 + i * BM, BM), :],
                    store_sems.at[slot])

            load(0, 0).start()

            def tile_body(i, carry):
                slot = lax.rem(i, 2)

                @pl.when(i + 1 < n_tiles)
                def _():
                    load(i + 1, lax.rem(i + 1, 2)).start()

                load(i, slot).wait()
                acc = jnp.dot(xt[slot], w_ref[...],
                              preferred_element_type=jnp.float32)

                @pl.when(i >= 2)
                def _():
                    store(i - 2, slot).wait()

                yt[slot] = jnp.maximum(acc, 0.0)
                store(i, slot).start()
                return carry

            lax.fori_loop(0, n_tiles, tile_body, 0)
            for j in range(max(0, n_tiles - 2), n_tiles):
                store(j, lax.rem(j, 2)).wait()

        rdma_a.start()
        rdma_b.start()
        gemm_chunk(x_ref, 0, me * M_PER)

        rdma_a.wait_recv()
        for r in rdma_c:
            r.start()
        gemm_chunk(xg_ref, 0, left * M_PER)

        rdma_b.wait_recv()
        for r in rdma_d:
            r.start()
        gemm_chunk(xg_ref, M_PER, right * M_PER)

        for j in range(HALF_M // SUB):
            rdma_c[j].wait_recv()
            gemm_chunk(xg_ref, 2 * M_PER + j * SUB,
                       diag * M_PER + j * SUB, n_tiles=SUB // BM)
            rdma_d[j].wait_recv()
            gemm_chunk(xg_ref, 2 * M_PER + HALF_M + j * SUB,
                       diag * M_PER + HALF_M + j * SUB, n_tiles=SUB // BM)

        rdma_a.wait_send()
        rdma_b.wait_send()
        for r in rdma_c + rdma_d:
            r.wait_send()

    y, _ = pl.pallas_call(
        body,
        out_shape=[
            jax.ShapeDtypeStruct((N_DEV * M_PER, N_PER), jnp.float32),
            jax.ShapeDtypeStruct((3 * M_PER, K), jnp.bfloat16),
        ],
        in_specs=[
            pl.BlockSpec(memory_space=pl.ANY),
            pl.BlockSpec(memory_space=pltpu.VMEM),
        ],
        out_specs=[
            pl.BlockSpec(memory_space=pl.ANY),
            pl.BlockSpec(memory_space=pl.ANY),
        ],
        scratch_shapes=[
            pltpu.VMEM((2, BM, K), jnp.bfloat16),
            pltpu.VMEM((2, BM, N_PER), jnp.float32),
            pltpu.SemaphoreType.DMA((2,)),
            pltpu.SemaphoreType.DMA((2,)),
            pltpu.SemaphoreType.DMA((6,)),
            pltpu.SemaphoreType.DMA((6,)),
        ],
        compiler_params=pltpu.CompilerParams(collective_id=0),
    )(x, w)
    return y


# device time: 233093 ns/iter; 3.0099x vs baseline; 2.8842x over previous
import os

import jax
import jax.numpy as jnp
from jax import lax
from jax.experimental import pallas as pl
from jax.experimental.pallas import tpu as pltpu

N_DEV = 4
M_PER = 2048
K = 8192
N_PER = 1024
HALF_M = M_PER // 2
BM = 256
SUB = 512
N_TILES = M_PER // BM
_SKIP_GEMM = bool(os.environ.get("SKIP_GEMM"))
_SKIP_COMM = bool(os.environ.get("SKIP_COMM"))


def kernel(x, w_mat):
    x = x.astype(jnp.bfloat16)
    w = w_mat.astype(jnp.bfloat16)

    def body(x_ref, w_ref, y_ref, xg_ref, xt, yt,
             load_sems, store_sems, send_sems, recv_sems):
        me = lax.axis_index("i")
        left = lax.rem(me + (N_DEV - 1), N_DEV)
        right = lax.rem(me + 1, N_DEV)
        diag = lax.rem(me + 2, N_DEV)

        barrier = pltpu.get_barrier_semaphore()
        pl.semaphore_signal(barrier, inc=1, device_id=(left,),
                            device_id_type=pl.DeviceIdType.MESH)
        pl.semaphore_signal(barrier, inc=1, device_id=(right,),
                            device_id_type=pl.DeviceIdType.MESH)
        pl.semaphore_wait(barrier, 2)

        def rdma(src, dst, i, dev):
            return pltpu.make_async_remote_copy(
                src_ref=src, dst_ref=dst,
                send_sem=send_sems.at[i], recv_sem=recv_sems.at[i],
                device_id=(dev,), device_id_type=pl.DeviceIdType.MESH,
            )

        slot0 = xg_ref.at[pl.ds(0, M_PER), :]
        slot1 = xg_ref.at[pl.ds(M_PER, M_PER), :]
        rdma_a = rdma(x_ref, slot0, 0, right)
        rdma_b = rdma(x_ref, slot1, 1, left)
        rdma_c = [
            rdma(xg_ref.at[pl.ds(j * SUB, SUB), :],
                 xg_ref.at[pl.ds(2 * M_PER + j * SUB, SUB), :],
                 2 + j, right)
            for j in range(HALF_M // SUB)
        ]
        rdma_d = [
            rdma(xg_ref.at[pl.ds(M_PER + HALF_M + j * SUB, SUB), :],
                 xg_ref.at[pl.ds(2 * M_PER + HALF_M + j * SUB, SUB), :],
                 2 + HALF_M // SUB + j, left)
            for j in range(HALF_M // SUB)
        ]

        def gemm_chunk(src_ref, src_row0, out_row0, n_tiles=N_TILES):
            if _SKIP_GEMM:
                return
            def load(i, slot):
                return pltpu.make_async_copy(
                    src_ref.at[pl.ds(src_row0 + i * BM, BM), :],
                    xt.at[slot], load_sems.at[slot])

            def store(i, slot):
                return pltpu.make_async_copy(
                    yt.at[slot],
                    y_ref.at[pl.ds(out_row0 + i * BM, BM), :],
                    store_sems.at[slot])

            load(0, 0).start()

            def tile_body(i, carry):
                slot = lax.rem(i, 2)

                @pl.when(i + 1 < n_tiles)
                def _():
                    load(i + 1, lax.rem(i + 1, 2)).start()

                load(i, slot).wait()
                acc = jnp.dot(xt[slot], w_ref[...],
                              preferred_element_type=jnp.float32)

                @pl.when(i >= 2)
                def _():
                    store(i - 2, slot).wait()

                yt[slot] = jnp.maximum(acc, 0.0)
                store(i, slot).start()
                return carry

            lax.fori_loop(0, n_tiles, tile_body, 0)
            for j in range(max(0, n_tiles - 2), n_tiles):
                store(j, lax.rem(j, 2)).wait()

        if _SKIP_COMM:
            gemm_chunk(x_ref, 0, me * M_PER)
            gemm_chunk(xg_ref, 0, left * M_PER)
            gemm_chunk(xg_ref, M_PER, right * M_PER)
            gemm_chunk(xg_ref, 2 * M_PER, diag * M_PER)
            return

        rdma_a.start()
        rdma_b.start()
        gemm_chunk(x_ref, 0, me * M_PER)

        rdma_a.wait_recv()
        for r in rdma_c:
            r.start()
        gemm_chunk(xg_ref, 0, left * M_PER)

        rdma_b.wait_recv()
        for r in rdma_d:
            r.start()
        gemm_chunk(xg_ref, M_PER, right * M_PER)

        for j in range(HALF_M // SUB):
            rdma_c[j].wait_recv()
            gemm_chunk(xg_ref, 2 * M_PER + j * SUB,
                       diag * M_PER + j * SUB, n_tiles=SUB // BM)
            rdma_d[j].wait_recv()
            gemm_chunk(xg_ref, 2 * M_PER + HALF_M + j * SUB,
                       diag * M_PER + HALF_M + j * SUB, n_tiles=SUB // BM)

        rdma_a.wait_send()
        rdma_b.wait_send()
        for r in rdma_c + rdma_d:
            r.wait_send()

    y, _ = pl.pallas_call(
        body,
        out_shape=[
            jax.ShapeDtypeStruct((N_DEV * M_PER, N_PER), jnp.float32),
            jax.ShapeDtypeStruct((3 * M_PER, K), jnp.bfloat16),
        ],
        in_specs=[
            pl.BlockSpec(memory_space=pl.ANY),
            pl.BlockSpec(memory_space=pltpu.VMEM),
        ],
        out_specs=[
            pl.BlockSpec(memory_space=pl.ANY),
            pl.BlockSpec(memory_space=pl.ANY),
        ],
        scratch_shapes=[
            pltpu.VMEM((2, BM, K), jnp.bfloat16),
            pltpu.VMEM((2, BM, N_PER), jnp.float32),
            pltpu.SemaphoreType.DMA((2,)),
            pltpu.SemaphoreType.DMA((2,)),
            pltpu.SemaphoreType.DMA((6,)),
            pltpu.SemaphoreType.DMA((6,)),
        ],
        compiler_params=pltpu.CompilerParams(collective_id=0),
    )(x, w)
    return y
